# baseline (speedup 1.0000x reference)
import jax
import jax.numpy as jnp
from jax import lax
from jax.experimental import pallas as pl
from jax.experimental.pallas import tpu as pltpu
from jax.experimental.pallas import tpu_sc as plsc

R, V = 128, 100000
NEG = -3.4e38

VS_TC = 76000
T = 20
CR = VS_TC // T
K = 6

NC, NS, L = 1, 16, 16
NW = NC * NS
VS_SC = V - VS_TC
SHARD = VS_SC // NW
CH = 125
NCH = SHARD // CH
G = R // L


def _tc_body(xt_hbm, o_ref, *scratch):
    bufs = scratch[:K]
    sems = scratch[K:]

    def issue(t):
        return pltpu.make_async_copy(
            xt_hbm.at[pl.ds(t * CR, CR), :], bufs[t % K], sems[t % K])

    cps = [issue(t) for t in range(K)]
    for cp in cps:
        cp.start()
    acc = jnp.full((R,), NEG, jnp.float32)
    for t in range(T):
        cps[t % K].wait()
        acc = jnp.maximum(acc, jnp.max(bufs[t % K][...], axis=0))
        if t + K < T:
            cps[t % K] = issue(t + K)
            cps[t % K].start()
    o_ref[0, :] = acc


def _sc_body(x_hbm, out_hbm, buf0, buf1, out_v, merge_v, shared, sem0, sem1):
    c = lax.axis_index("c")
    s = lax.axis_index("s")
    wid = s * NC + c
    base = VS_TC * R + wid * SHARD * R
    bufs = (buf0, buf1)
    sems = (sem0, sem1)

    def issue(k):
        off = pl.multiple_of(base + k * CH * R, 8)
        return pltpu.async_copy(
            x_hbm.at[pl.ds(off, CH * R)], bufs[k % 2], sems[k % 2])

    cps = [issue(0), None]
    accs = tuple(jnp.full((L,), NEG, jnp.float32) for _ in range(G))
    for k in range(NCH):
        if k + 1 < NCH:
            cps[(k + 1) % 2] = issue(k + 1)
        cps[k % 2].wait()
        buf = bufs[k % 2]

        def body(i, a, buf=buf):
            out = list(a)
            for g in range(G):
                out[g] = jnp.maximum(out[g], buf[pl.ds(i * R + g * L, L)])
            return tuple(out)

        accs = lax.fori_loop(0, CH, body, accs)

    for g in range(G):
        out_v[pl.ds(g * L, L)] = accs[g]
    pltpu.sync_copy(out_v, shared.at[pl.ds(s * R, R)])
    plsc.subcore_barrier()

    @pl.when(s == 0)
    def _():
        pltpu.sync_copy(shared, merge_v)
        m = tuple(merge_v[pl.ds(g * L, L)] for g in range(G))

        def mbody(i, mm):
            out = list(mm)
            for g in range(G):
                out[g] = jnp.maximum(out[g], merge_v[pl.ds(i * R + g * L, L)])
            return tuple(out)

        mm = lax.fori_loop(1, NS, mbody, m)
        for g in range(G):
            out_v[pl.ds(g * L, L)] = mm[g]
        pltpu.sync_copy(out_v, out_hbm.at[pl.ds(c * R, R)])


def kernel(X):
    Xt = X.T
    tc_out = pl.pallas_call(
        _tc_body,
        in_specs=[pl.BlockSpec(memory_space=pl.ANY)],
        out_specs=pl.BlockSpec(memory_space=pltpu.MemorySpace.VMEM),
        out_shape=jax.ShapeDtypeStruct((1, R), jnp.float32),
        scratch_shapes=(
            [pltpu.VMEM((CR, R), jnp.float32) for _ in range(K)]
            + [pltpu.SemaphoreType.DMA for _ in range(K)]
        ),
    )(Xt)

    sc_call = pl.kernel(
        _sc_body,
        out_type=jax.ShapeDtypeStruct((NC * R,), jnp.float32),
        mesh=plsc.VectorSubcoreMesh(core_axis_name="c", subcore_axis_name="s", num_cores=NC),
        scratch_types=[
            pltpu.VMEM((CH * R,), jnp.float32),
            pltpu.VMEM((CH * R,), jnp.float32),
            pltpu.VMEM((R,), jnp.float32),
            pltpu.VMEM((NS * R,), jnp.float32),
            pltpu.VMEM_SHARED((NS * R,), jnp.float32),
            pltpu.SemaphoreType.DMA,
            pltpu.SemaphoreType.DMA,
        ],
        compiler_params=pltpu.CompilerParams(needs_layout_passes=False),
    )
    sc_pair = sc_call(Xt.reshape(-1)).reshape(NC, R)

    out = tc_out[0]
    for i in range(NC):
        out = jnp.maximum(out, sc_pair[i])
    return out


# speedup vs baseline: 1.0340x; 1.0340x over previous
import jax
import jax.numpy as jnp
from jax import lax
from jax.experimental import pallas as pl
from jax.experimental.pallas import tpu as pltpu
from jax.experimental.pallas import tpu_sc as plsc

R, V = 128, 100000
NEG = -3.4e38

VS_TC = 76000
T = 20
CR = VS_TC // T
K = 6

NC, NS, L = 1, 16, 16
NW = NC * NS
VS_SC = V - VS_TC
SHARD = VS_SC // NW
CH = 125
NCH = SHARD // CH
G = R // L


def _tc_body(xt_hbm, o_ref, *scratch):
    bufs = scratch[:K]
    sems = scratch[K:]

    def issue(t):
        return pltpu.make_async_copy(
            xt_hbm.at[pl.ds(t * CR, CR), :], bufs[t % K], sems[t % K])

    cps = [issue(t) for t in range(K)]
    for cp in cps:
        cp.start()
    acc = jnp.full((R,), NEG, jnp.float32)
    for t in range(T):
        cps[t % K].wait()
        acc = jnp.maximum(acc, jnp.max(bufs[t % K][...], axis=0))
        if t + K < T:
            cps[t % K] = issue(t + K)
            cps[t % K].start()
    o_ref[0, :] = acc


def _sc_body(x_hbm, out_hbm, buf0, buf1, out_v, merge_v, shared, sem0, sem1):
    c = lax.axis_index("c")
    s = lax.axis_index("s")
    wid = s * NC + c
    base = VS_TC * R + wid * SHARD * R
    bufs = (buf0, buf1)
    sems = (sem0, sem1)

    def issue(k):
        off = pl.multiple_of(base + k * CH * R, 8)
        return pltpu.async_copy(
            x_hbm.at[pl.ds(off, CH * R)], bufs[k % 2], sems[k % 2])

    cps = [issue(0), None]
    accs = tuple(jnp.full((L,), NEG, jnp.float32) for _ in range(G))
    for k in range(NCH):
        if k + 1 < NCH:
            cps[(k + 1) % 2] = issue(k + 1)
        cps[k % 2].wait()
        buf = bufs[k % 2]

        def body(i, a, buf=buf):
            out = list(a)
            for g in range(G):
                out[g] = jnp.maximum(out[g], buf[pl.ds(i * R + g * L, L)])
            return tuple(out)

        accs = lax.fori_loop(0, CH, body, accs)

    for g in range(G):
        out_v[pl.ds(g * L, L)] = accs[g]
    pltpu.sync_copy(out_v, shared.at[pl.ds(s * R, R)])
    plsc.subcore_barrier()

    @pl.when(s == 0)
    def _():
        pltpu.sync_copy(shared, merge_v)
        m = tuple(merge_v[pl.ds(g * L, L)] for g in range(G))

        def mbody(i, mm):
            out = list(mm)
            for g in range(G):
                out[g] = jnp.maximum(out[g], merge_v[pl.ds(i * R + g * L, L)])
            return tuple(out)

        mm = lax.fori_loop(1, NS, mbody, m)
        for g in range(G):
            out_v[pl.ds(g * L, L)] = mm[g]
        pltpu.sync_copy(out_v, out_hbm.at[pl.ds(c * R, R)])


def kernel(X):
    Xt = X.T
    sc_call = pl.kernel(
        _sc_body,
        out_type=jax.ShapeDtypeStruct((NC * R,), jnp.float32),
        mesh=plsc.VectorSubcoreMesh(core_axis_name="c", subcore_axis_name="s", num_cores=NC),
        scratch_types=[
            pltpu.VMEM((CH * R,), jnp.float32),
            pltpu.VMEM((CH * R,), jnp.float32),
            pltpu.VMEM((R,), jnp.float32),
            pltpu.VMEM((NS * R,), jnp.float32),
            pltpu.VMEM_SHARED((NS * R,), jnp.float32),
            pltpu.SemaphoreType.DMA,
            pltpu.SemaphoreType.DMA,
        ],
        compiler_params=pltpu.CompilerParams(needs_layout_passes=False),
    )
    sc_pair = sc_call(Xt.reshape(-1)).reshape(NC, R)

    tc_out = pl.pallas_call(
        _tc_body,
        in_specs=[pl.BlockSpec(memory_space=pl.ANY)],
        out_specs=pl.BlockSpec(memory_space=pltpu.MemorySpace.VMEM),
        out_shape=jax.ShapeDtypeStruct((1, R), jnp.float32),
        scratch_shapes=(
            [pltpu.VMEM((CR, R), jnp.float32) for _ in range(K)]
            + [pltpu.SemaphoreType.DMA for _ in range(K)]
        ),
    )(Xt)

    out = tc_out[0]
    for i in range(NC):
        out = jnp.maximum(out, sc_pair[i])
    return out


# speedup vs baseline: 2.1043x; 2.0352x over previous
import jax
import jax.numpy as jnp
from jax.experimental import pallas as pl
from jax.experimental.pallas import tpu as pltpu

R, V = 128, 100000
T = 25
CR = V // T
K = 8

NEG = -3.4e38


def _max_body(xt_hbm, o_ref, *scratch):
    bufs = scratch[:K]
    sems = scratch[K:]

    def issue(t):
        return pltpu.make_async_copy(
            xt_hbm.at[pl.ds(t * CR, CR), :], bufs[t % K], sems[t % K])

    cps = [issue(t) for t in range(K)]
    for cp in cps:
        cp.start()
    acc = jnp.full((R,), NEG, jnp.float32)
    for t in range(T):
        cps[t % K].wait()
        acc = jnp.maximum(acc, jnp.max(bufs[t % K][...], axis=0))
        if t + K < T:
            cps[t % K] = issue(t + K)
            cps[t % K].start()
    o_ref[0, :] = acc


def kernel(X):
    out = pl.pallas_call(
        _max_body,
        in_specs=[pl.BlockSpec(memory_space=pl.ANY)],
        out_specs=pl.BlockSpec(memory_space=pltpu.MemorySpace.VMEM),
        out_shape=jax.ShapeDtypeStruct((1, R), jnp.float32),
        scratch_shapes=(
            [pltpu.VMEM((CR, R), jnp.float32) for _ in range(K)]
            + [pltpu.SemaphoreType.DMA for _ in range(K)]
        ),
    )(X.T)
    return out[0]


# speedup vs baseline: 2.2100x; 1.0502x over previous
import jax
import jax.numpy as jnp
from jax.experimental import pallas as pl
from jax.experimental.pallas import tpu as pltpu

R, V = 128, 100000
T = 20
CR = V // T
K = 8

NEG = -3.4e38


def _max_body(xt_hbm, o_ref, *scratch):
    bufs = scratch[:K]
    sems = scratch[K:]

    def issue(t):
        return pltpu.make_async_copy(
            xt_hbm.at[pl.ds(t * CR, CR), :], bufs[t % K], sems[t % K])

    cps = [issue(t) for t in range(K)]
    for cp in cps:
        cp.start()
    acc = jnp.full((R,), NEG, jnp.float32)
    for t in range(T):
        cps[t % K].wait()
        acc = jnp.maximum(acc, jnp.max(bufs[t % K][...], axis=0))
        if t + K < T:
            cps[t % K] = issue(t + K)
            cps[t % K].start()
    o_ref[0, :] = acc


def kernel(X):
    out = pl.pallas_call(
        _max_body,
        in_specs=[pl.BlockSpec(memory_space=pl.ANY)],
        out_specs=pl.BlockSpec(memory_space=pltpu.MemorySpace.VMEM),
        out_shape=jax.ShapeDtypeStruct((1, R), jnp.float32),
        scratch_shapes=(
            [pltpu.VMEM((CR, R), jnp.float32) for _ in range(K)]
            + [pltpu.SemaphoreType.DMA for _ in range(K)]
        ),
    )(X.T)
    return out[0]
